# baseline (speedup 1.0000x reference)
import functools

import jax
import jax.numpy as jnp
from jax import lax
from jax.experimental import pallas as pl
from jax.experimental.pallas import tpu as pltpu
from jax.experimental.pallas import tpu_sc as plsc

_NC = 2
_NS = 16
_NW = _NC * _NS
_B = 16384
_CHUNK = _B // _NW
_STEPS = _CHUNK // 16

_mesh = plsc.VectorSubcoreMesh(core_axis_name="c", subcore_axis_name="s")


@functools.partial(
    pl.kernel,
    out_type=jax.ShapeDtypeStruct((_B * 4,), jnp.float32),
    mesh=_mesh,
    compiler_params=pltpu.CompilerParams(needs_layout_passes=False),
    scratch_types=[
        pltpu.VMEM((_CHUNK,), jnp.int32),
        pltpu.VMEM((_CHUNK,), jnp.int32),
        pltpu.VMEM((3, 2), jnp.float32),
        pltpu.VMEM((3, 2), jnp.float32),
        pltpu.VMEM((_CHUNK * 4,), jnp.float32),
        pltpu.SemaphoreType.DMA,
    ],
)
def _encode(y_hbm, s_hbm, e1_hbm, e2_hbm, out_hbm, y_v, s_v, e1_v, e2_v, o_v,
            sem):
    wid = lax.axis_index("s") * _NC + lax.axis_index("c")
    base = wid * _CHUNK
    c1 = pltpu.async_copy(e1_hbm, e1_v, sem)
    c2 = pltpu.async_copy(e2_hbm, e2_v, sem)
    c3 = pltpu.async_copy(y_hbm.at[pl.ds(base, _CHUNK)], y_v, sem)
    c4 = pltpu.async_copy(s_hbm.at[pl.ds(base, _CHUNK)], s_v, sem)
    c1.wait()
    c2.wait()
    c3.wait()
    c4.wait()

    lanes = lax.iota(jnp.int32, 16)
    zero = lanes * 0
    one = zero + 1
    for j in range(_STEPS):
        y16 = y_v[pl.ds(j * 16, 16)]
        s16 = s_v[pl.ds(j * 16, 16)]
        a = plsc.load_gather(e1_v, [y16, zero])
        b = plsc.load_gather(e1_v, [y16, one])
        c = plsc.load_gather(e2_v, [s16, zero])
        d = plsc.load_gather(e2_v, [s16, one])
        p = (j * 16 + lanes) * 4
        plsc.store_scatter(o_v, [p], a)
        plsc.store_scatter(o_v, [p + 1], b)
        plsc.store_scatter(o_v, [p + 2], c)
        plsc.store_scatter(o_v, [p + 3], d)

    pltpu.sync_copy(o_v, out_hbm.at[pl.ds(base * 4, _CHUNK * 4)])


def kernel(y, s, emb1_w, emb2_w):
    return _encode(y, s, emb1_w, emb2_w).reshape(_B, 4)
